# Initial kernel scaffold; baseline (speedup 1.0000x reference)
import jax
import jax.numpy as jnp
from jax.experimental import pallas as pl


def kernel(pos, batch):
    raise NotImplementedError("write your pallas kernel here")


# speedup vs baseline: 7.8706x; 7.8706x over previous
import functools

import jax
import jax.numpy as jnp
from jax import lax
from jax.experimental import pallas as pl

N = 4096
K = 32
CUTOFF2 = 2.5 * 2.5
ROWS = 256


def _topk_kernel(pos_row_ref, bat_row_ref, pos_t_ref, bat_t_ref,
                 idx_ref, w_ref):
    r = pl.program_id(0)
    c = pos_t_ref.shape[1]

    xi = pos_row_ref[:, 0:1]
    yi = pos_row_ref[:, 1:2]
    zi = pos_row_ref[:, 2:3]
    xj = pos_t_ref[0:1, :]
    yj = pos_t_ref[1:2, :]
    zj = pos_t_ref[2:3, :]

    sqi = xi * xi + yi * yi + zi * zi
    sqj = xj * xj + yj * yj + zj * zj
    dot = jnp.dot(pos_row_ref[:, :], pos_t_ref[:, :],
                  preferred_element_type=jnp.float32)
    d2 = (sqi + sqj) - 2.0 * dot
    d2 = jnp.maximum(d2, 0.0)

    dx = xi - xj
    dy = yi - yj
    dz = zi - zj
    d2e = dx * dx + dy * dy + dz * dz

    iota_j = lax.broadcasted_iota(jnp.int32, (ROWS, c), 1)
    row_ids = r * ROWS + lax.broadcasted_iota(jnp.int32, (ROWS, 1), 0)
    same = bat_row_ref[:, 0:1] == bat_t_ref[0:1, :]
    valid = same & (iota_j != row_ids) & (d2 <= CUTOFF2)

    inf = jnp.float32(jnp.inf)
    key = jnp.where(valid, d2, inf)
    iota_f = iota_j.astype(jnp.float32)
    big = jnp.float32(N)

    row_ids_f = row_ids.astype(jnp.float32)
    for k in range(K):
        m = jnp.min(key, axis=1, keepdims=True)
        hit = key == m
        idxm = jnp.min(jnp.where(hit, iota_f, big), axis=1, keepdims=True)
        sel = iota_f == idxm
        w2 = jnp.min(jnp.where(sel, d2e, inf), axis=1, keepdims=True)
        finite = m < inf
        idx_ref[:, k:k + 1] = jnp.where(finite, idxm, row_ids_f).astype(jnp.int32)
        w_ref[:, k:k + 1] = jnp.where(finite, jnp.sqrt(w2), 0.0)
        key = jnp.where(sel, inf, key)


@jax.jit
def kernel(pos, batch):
    bat32 = batch.astype(jnp.int32)
    pos_t = pos.T
    bat_row = bat32.reshape(N, 1)
    bat_t = bat32.reshape(1, N)

    grid = (N // ROWS,)
    idx, w = pl.pallas_call(
        _topk_kernel,
        grid=grid,
        in_specs=[
            pl.BlockSpec((ROWS, 3), lambda r: (r, 0)),
            pl.BlockSpec((ROWS, 1), lambda r: (r, 0)),
            pl.BlockSpec((3, N), lambda r: (0, 0)),
            pl.BlockSpec((1, N), lambda r: (0, 0)),
        ],
        out_specs=[
            pl.BlockSpec((ROWS, K), lambda r: (r, 0)),
            pl.BlockSpec((ROWS, K), lambda r: (r, 0)),
        ],
        out_shape=[
            jax.ShapeDtypeStruct((N, K), jnp.int32),
            jax.ShapeDtypeStruct((N, K), jnp.float32),
        ],
    )(pos, bat_row, pos_t, bat_t)

    tgt = jnp.broadcast_to(jnp.arange(N, dtype=jnp.int32)[:, None], (N, K))
    edge_index = jnp.stack([idx.reshape(-1), tgt.reshape(-1)]).astype(jnp.int64)
    edge_weight = w.reshape(-1)
    return edge_index, edge_weight


# speedup vs baseline: 22.0226x; 2.7981x over previous
import functools

import jax
import jax.numpy as jnp
from jax import lax
from jax.experimental import pallas as pl
from jax.experimental.pallas import tpu as pltpu

N = 4096
K = 32
CUTOFF2 = 2.5 * 2.5
ROWS = 256
WIN = 1536


def _topk_kernel(w_ref, pos_row_ref, bat_row_ref, pos_t_ref, bat_t_ref,
                 idx_ref, wout_ref, *, width):
    r = pl.program_id(0)
    w0 = pl.multiple_of(w_ref[r], 128)

    xi = pos_row_ref[:, 0:1]
    yi = pos_row_ref[:, 1:2]
    zi = pos_row_ref[:, 2:3]
    pt = pos_t_ref[:, pl.ds(w0, width)]
    xj = pt[0:1, :]
    yj = pt[1:2, :]
    zj = pt[2:3, :]

    sqi = xi * xi + yi * yi + zi * zi
    sqj = xj * xj + yj * yj + zj * zj
    dot = jnp.dot(pos_row_ref[:, :], pt,
                  preferred_element_type=jnp.float32)
    d2 = (sqi + sqj) - 2.0 * dot
    d2 = jnp.maximum(d2, 0.0)

    dx = xi - xj
    dy = yi - yj
    dz = zi - zj
    d2e = dx * dx + dy * dy + dz * dz

    iota_j = w0 + lax.broadcasted_iota(jnp.int32, (ROWS, width), 1)
    row_ids = r * ROWS + lax.broadcasted_iota(jnp.int32, (ROWS, 1), 0)
    same = bat_row_ref[:, 0:1] == bat_t_ref[0:1, pl.ds(w0, width)]
    valid = same & (iota_j != row_ids) & (d2 <= CUTOFF2)

    inf = jnp.float32(jnp.inf)
    key = jnp.where(valid, d2, inf)
    iota_f = iota_j.astype(jnp.float32)
    big = jnp.float32(N)

    row_ids_f = row_ids.astype(jnp.float32)
    for k in range(K):
        m = jnp.min(key, axis=1, keepdims=True)
        hit = key == m
        idxm = jnp.min(jnp.where(hit, iota_f, big), axis=1, keepdims=True)
        sel = iota_f == idxm
        w2 = jnp.min(jnp.where(sel, d2e, inf), axis=1, keepdims=True)
        finite = m < inf
        idx_ref[:, k:k + 1] = jnp.where(finite, idxm, row_ids_f).astype(jnp.int32)
        wout_ref[:, k:k + 1] = jnp.where(finite, jnp.sqrt(w2), 0.0)
        key = jnp.where(sel, inf, key)


def _call(width, wstarts, pos, bat_row, pos_t, bat_t):
    grid_spec = pltpu.PrefetchScalarGridSpec(
        num_scalar_prefetch=1,
        grid=(N // ROWS,),
        in_specs=[
            pl.BlockSpec((ROWS, 3), lambda r, w: (r, 0)),
            pl.BlockSpec((ROWS, 1), lambda r, w: (r, 0)),
            pl.BlockSpec((3, N), lambda r, w: (0, 0)),
            pl.BlockSpec((1, N), lambda r, w: (0, 0)),
        ],
        out_specs=[
            pl.BlockSpec((ROWS, K), lambda r, w: (r, 0)),
            pl.BlockSpec((ROWS, K), lambda r, w: (r, 0)),
        ],
    )
    return pl.pallas_call(
        functools.partial(_topk_kernel, width=width),
        grid_spec=grid_spec,
        out_shape=[
            jax.ShapeDtypeStruct((N, K), jnp.int32),
            jax.ShapeDtypeStruct((N, K), jnp.float32),
        ],
    )(wstarts, pos, bat_row, pos_t, bat_t)


@jax.jit
def kernel(pos, batch):
    bat32 = batch.astype(jnp.int32)
    pos_t = pos.T
    bat_row = bat32.reshape(N, 1)
    bat_t = bat32.reshape(1, N)

    t0 = jnp.arange(N // ROWS, dtype=jnp.int32) * ROWS
    g_lo = bat32[t0]
    g_hi = bat32[t0 + (ROWS - 1)]
    col_lo = jnp.searchsorted(bat32, g_lo, side="left").astype(jnp.int32)
    col_hi = jnp.searchsorted(bat32, g_hi, side="right").astype(jnp.int32)
    wstarts = jnp.minimum((col_lo // 128) * 128, N - WIN)
    fits = jnp.max(col_hi - wstarts) <= WIN
    zeros = jnp.zeros_like(wstarts)

    idx, w = lax.cond(
        fits,
        lambda: _call(WIN, wstarts, pos, bat_row, pos_t, bat_t),
        lambda: _call(N, zeros, pos, bat_row, pos_t, bat_t),
    )

    tgt = jnp.broadcast_to(jnp.arange(N, dtype=jnp.int32)[:, None], (N, K))
    edge_index = jnp.stack([idx.reshape(-1), tgt.reshape(-1)]).astype(jnp.int64)
    edge_weight = w.reshape(-1)
    return edge_index, edge_weight
